# baseline (speedup 1.0000x reference)
import functools

import jax
import jax.numpy as jnp
from jax import lax
from jax.experimental import pallas as pl
from jax.experimental.pallas import tpu as pltpu
from jax.experimental.pallas import tpu_sc as plsc

N_NODES = 10000
N_EDGES = 320000
D = 128
NUM_RELS = 64
NUM_BASES = 4

NC = 2
NS = 16
NW = NC * NS

E_HALF = N_EDGES // 2

E_BLK = 6400
N_BLK = 2048

C_GA = 200
C_SC = 40
N_PAD = 10240
NPT = N_PAD // NS
ZR = 128


@functools.cache
def _gather_kernel(n_e, c_ga):
    epw = n_e // NW
    nchunks = epw // c_ga

    def body(x_hbm, src_hbm, out_hbm,
             idx0, idx1, rows0, rows1, gsem0, gsem1, osem0, osem1):
        wid = lax.axis_index("s") * NC + lax.axis_index("c")
        base = wid * epw
        idx = (idx0, idx1)
        rows = (rows0, rows1)
        gsem = (gsem0, gsem1)
        osem = (osem0, osem1)

        pltpu.sync_copy(src_hbm.at[pl.ds(base, c_ga)], idx0)
        gathers = [pltpu.async_copy(x_hbm.at[idx0], rows0, gsem0)]
        outs = [None] * nchunks
        for k in range(nchunks):
            s = k % 2
            if k + 1 < nchunks:
                o = base + (k + 1) * c_ga
                pltpu.sync_copy(src_hbm.at[pl.ds(o, c_ga)], idx[1 - s])
                if k >= 1:
                    outs[k - 1].wait()
                gathers.append(
                    pltpu.async_copy(x_hbm.at[idx[1 - s]], rows[1 - s], gsem[1 - s]))
            gathers[k].wait()
            outs[k] = pltpu.async_copy(
                rows[s], out_hbm.at[pl.ds(base + k * c_ga, c_ga)], osem[s])
        if nchunks >= 2:
            outs[nchunks - 2].wait()
        outs[nchunks - 1].wait()

    return pl.kernel(
        body,
        out_type=jax.ShapeDtypeStruct((n_e, D), jnp.float32),
        mesh=plsc.VectorSubcoreMesh(core_axis_name="c", subcore_axis_name="s",
                                    num_cores=NC, num_subcores=NS),
        scratch_types=[
            pltpu.VMEM((c_ga,), jnp.int32),
            pltpu.VMEM((c_ga,), jnp.int32),
            pltpu.VMEM((c_ga, D), jnp.float32),
            pltpu.VMEM((c_ga, D), jnp.float32),
            pltpu.SemaphoreType.DMA,
            pltpu.SemaphoreType.DMA,
            pltpu.SemaphoreType.DMA,
            pltpu.SemaphoreType.DMA,
        ],
    )


@functools.cache
def _scatter_kernel(n_e, c_sc):
    epw = n_e // NW
    nchunks = epw // c_sc

    def body(msg_hbm, dst_hbm, out_hbm,
             idx0, idx1, msg0, msg1, msem0, msem1, acc_sh):
        c = lax.axis_index("c")
        s = lax.axis_index("s")
        wid = s * NC + c
        base = wid * epw
        idx = (idx0, idx1)
        msg = (msg0, msg1)
        msem = (msem0, msem1)

        def zrow(r, carry):
            for k in range(D // 16):
                msg0[r, pl.ds(k * 16, 16)] = jnp.zeros((16,), jnp.float32)
            return carry

        lax.fori_loop(0, c_sc, zrow, 0)
        for t in range(NPT // c_sc):
            pltpu.sync_copy(msg0.at[pl.ds(0, c_sc)],
                            acc_sh.at[pl.ds(s * NPT + t * c_sc, c_sc)])
        plsc.subcore_barrier()

        pltpu.sync_copy(dst_hbm.at[pl.ds(base, c_sc)], idx0)
        loads = [pltpu.async_copy(msg_hbm.at[pl.ds(base, c_sc)], msg0, msem0)]
        for k in range(nchunks):
            sl = k % 2
            if k + 1 < nchunks:
                o = base + (k + 1) * c_sc
                pltpu.sync_copy(dst_hbm.at[pl.ds(o, c_sc)], idx[1 - sl])
                loads.append(
                    pltpu.async_copy(msg_hbm.at[pl.ds(o, c_sc)], msg[1 - sl],
                                     msem[1 - sl]))
            loads[k].wait()
            pltpu.sync_copy(msg[sl], acc_sh.at[idx[sl]], add=True)
        plsc.subcore_barrier()

        for t in range(NPT // c_sc):
            row = s * NPT + t * c_sc
            pltpu.sync_copy(acc_sh.at[pl.ds(row, c_sc)], msg0.at[pl.ds(0, c_sc)])
            pltpu.sync_copy(msg0.at[pl.ds(0, c_sc)], out_hbm.at[c, pl.ds(row, c_sc)])

    return pl.kernel(
        body,
        out_type=jax.ShapeDtypeStruct((NC, N_PAD, D), jnp.float32),
        mesh=plsc.VectorSubcoreMesh(core_axis_name="c", subcore_axis_name="s",
                                    num_cores=NC, num_subcores=NS),
        scratch_types=[
            pltpu.VMEM((c_sc,), jnp.int32),
            pltpu.VMEM((c_sc,), jnp.int32),
            pltpu.VMEM((c_sc, D), jnp.float32),
            pltpu.VMEM((c_sc, D), jnp.float32),
            pltpu.SemaphoreType.DMA,
            pltpu.SemaphoreType.DMA,
            pltpu.VMEM_SHARED((N_PAD, D), jnp.float32),
        ],
    )


def _msg_body(et_ref, xsrc_ref, relw_ref, wflat_ref, msg_ref):
    et = et_ref[0, 0, :]
    onehot = (et[:, None] == lax.broadcasted_iota(jnp.int32, (E_BLK, NUM_RELS), 1)).astype(jnp.bfloat16)
    edw = jnp.dot(onehot, relw_ref[...], preferred_element_type=jnp.float32)
    x4 = jnp.concatenate([xsrc_ref[...]] * NUM_BASES, axis=1).astype(jnp.float32)
    ed4 = (x4 * edw).astype(jnp.bfloat16)
    msg_ref[...] = jnp.dot(ed4, wflat_ref[...], preferred_element_type=jnp.float32)


def _edge_messages(et3, xsrc, relw, wflat):
    n_e = xsrc.shape[0]
    nblk = n_e // E_BLK
    return pl.pallas_call(
        _msg_body,
        grid=(nblk,),
        in_specs=[
            pl.BlockSpec((1, 1, E_BLK), lambda i: (i, 0, 0)),
            pl.BlockSpec((E_BLK, D), lambda i: (i, 0)),
            pl.BlockSpec((NUM_RELS, NUM_BASES * D), lambda i: (0, 0)),
            pl.BlockSpec((NUM_BASES * D, D), lambda i: (0, 0)),
        ],
        out_specs=pl.BlockSpec((E_BLK, D), lambda i: (i, 0)),
        out_shape=jax.ShapeDtypeStruct((n_e, D), jnp.float32),
    )(et3, xsrc, relw, wflat)


def _node_body(part_ref, x_ref, wself_ref, rel_ref, wrel_ref, out_ref, relout_ref):
    i = pl.program_id(0)
    agg = jnp.sum(part_ref[...], axis=0)
    out_ref[...] = jnp.maximum(
        agg + jnp.dot(x_ref[...], wself_ref[...], preferred_element_type=jnp.float32), 0.0)

    @pl.when(i == 0)
    def _():
        ro = jnp.dot(rel_ref[...], wrel_ref[...], preferred_element_type=jnp.float32)
        row = lax.broadcasted_iota(jnp.int32, ro.shape, 0)
        relout_ref[...] = jnp.where(row == ro.shape[0] - 1, 0.0, ro)


def _node_update(partials, x, self_loop_weight, rel_emb, w_rel):
    nblk = (N_NODES + N_BLK - 1) // N_BLK
    p = partials.shape[0]
    return pl.pallas_call(
        _node_body,
        grid=(nblk,),
        in_specs=[
            pl.BlockSpec((p, N_BLK, D), lambda i: (0, i, 0)),
            pl.BlockSpec((N_BLK, D), lambda i: (i, 0)),
            pl.BlockSpec((D, D), lambda i: (0, 0)),
            pl.BlockSpec((NUM_RELS + 1, D), lambda i: (0, 0)),
            pl.BlockSpec((D, D), lambda i: (0, 0)),
        ],
        out_specs=[
            pl.BlockSpec((N_BLK, D), lambda i: (i, 0)),
            pl.BlockSpec((NUM_RELS + 1, D), lambda i: (0, 0)),
        ],
        out_shape=[
            jax.ShapeDtypeStruct((N_NODES, D), jnp.float32),
            jax.ShapeDtypeStruct((NUM_RELS + 1, D), jnp.float32),
        ],
    )(partials, x, self_loop_weight, rel_emb, w_rel)


def kernel(x, edge_index, edge_type, rel_emb, weight_basis, w_comp, self_loop_weight, w_rel):
    src = edge_index[0].astype(jnp.int32)
    dst = edge_index[1].astype(jnp.int32)
    et = edge_type.astype(jnp.int32)
    relw = (w_comp[:, :, None] * rel_emb[None, :NUM_RELS, :].transpose(1, 0, 2)).reshape(
        NUM_RELS, NUM_BASES * D).astype(jnp.bfloat16)
    wflat = weight_basis.reshape(NUM_BASES * D, D).astype(jnp.bfloat16)

    gather = _gather_kernel(E_HALF, C_GA)
    scatter = _scatter_kernel(E_HALF, C_SC)

    halves = []
    for h in range(2):
        sl = slice(h * E_HALF, (h + 1) * E_HALF)
        et3 = et[sl].reshape(E_HALF // E_BLK, 1, E_BLK)
        xsrc = gather(x, src[sl])
        msg = _edge_messages(et3, xsrc, relw, wflat)
        halves.append(scatter(msg, dst[sl]))

    partials = jnp.concatenate(halves, axis=0)
    node_repr, rel_out = _node_update(partials, x, self_loop_weight, rel_emb, w_rel)
    return node_repr, rel_out


# speedup vs baseline: 1.1749x; 1.1749x over previous
import functools

import jax
import jax.numpy as jnp
from jax import lax
from jax.experimental import pallas as pl
from jax.experimental.pallas import tpu as pltpu
from jax.experimental.pallas import tpu_sc as plsc

N_NODES = 10000
N_EDGES = 320000
D = 128
NUM_RELS = 64
NUM_BASES = 4

NC = 2
NS = 16
NW = NC * NS

E_HALF = N_EDGES // 2

E_BLK = 6400
N_BLK = 2048

C_GA = 200
C_SC = 200
N_PAD = 10240
NPT = N_PAD // NS
ZR = 128


@functools.cache
def _gather_kernel(n_e, c_ga):
    epw = n_e // NW
    nchunks = epw // c_ga

    def body(x_hbm, src_hbm, out_hbm,
             idx_all, rows0, rows1, gsem0, gsem1, osem0, osem1):
        wid = lax.axis_index("s") * NC + lax.axis_index("c")
        base = wid * epw
        rows = (rows0, rows1)
        gsem = (gsem0, gsem1)
        osem = (osem0, osem1)

        pltpu.sync_copy(src_hbm.at[pl.ds(base, epw)], idx_all)

        gathers = [pltpu.async_copy(
            x_hbm.at[idx_all.at[pl.ds(0, c_ga)]], rows0, gsem0)]
        outs = [None] * nchunks
        for k in range(nchunks):
            s = k % 2
            if k + 1 < nchunks:
                if k >= 1:
                    outs[k - 1].wait()
                gathers.append(pltpu.async_copy(
                    x_hbm.at[idx_all.at[pl.ds((k + 1) * c_ga, c_ga)]],
                    rows[1 - s], gsem[1 - s]))
            gathers[k].wait()
            outs[k] = pltpu.async_copy(
                rows[s], out_hbm.at[pl.ds(base + k * c_ga, c_ga)], osem[s])
        if nchunks >= 2:
            outs[nchunks - 2].wait()
        outs[nchunks - 1].wait()

    return pl.kernel(
        body,
        out_type=jax.ShapeDtypeStruct((n_e, D), jnp.float32),
        mesh=plsc.VectorSubcoreMesh(core_axis_name="c", subcore_axis_name="s",
                                    num_cores=NC, num_subcores=NS),
        scratch_types=[
            pltpu.VMEM((epw,), jnp.int32),
            pltpu.VMEM((c_ga, D), jnp.float32),
            pltpu.VMEM((c_ga, D), jnp.float32),
            pltpu.SemaphoreType.DMA,
            pltpu.SemaphoreType.DMA,
            pltpu.SemaphoreType.DMA,
            pltpu.SemaphoreType.DMA,
        ],
    )


@functools.cache
def _scatter_kernel(n_e, c_sc):
    epw = n_e // NW
    nchunks = epw // c_sc

    def body(msg_hbm, dst_hbm, out_hbm, idx_all, msg_v, msem, acc_sh):
        c = lax.axis_index("c")
        s = lax.axis_index("s")
        wid = s * NC + c
        base = wid * epw

        idx_load = pltpu.async_copy(dst_hbm.at[pl.ds(base, epw)], idx_all, msem)

        def zrow(r, carry):
            for k in range(D // 16):
                msg_v[r, pl.ds(k * 16, 16)] = jnp.zeros((16,), jnp.float32)
            return carry

        lax.fori_loop(0, ZR, zrow, 0)
        for t in range(NPT // ZR):
            pltpu.sync_copy(msg_v.at[pl.ds(0, ZR)],
                            acc_sh.at[pl.ds(s * NPT + t * ZR, ZR)])
        idx_load.wait()
        plsc.subcore_barrier()

        def chunk(k, carry):
            off = base + k * c_sc
            pltpu.sync_copy(msg_hbm.at[pl.ds(off, c_sc)], msg_v)
            pltpu.sync_copy(msg_v, acc_sh.at[idx_all.at[pl.ds(k * c_sc, c_sc)]],
                            add=True)
            return carry

        lax.fori_loop(0, nchunks, chunk, 0)
        plsc.subcore_barrier()

        for t in range(NPT // ZR):
            row = s * NPT + t * ZR
            pltpu.sync_copy(acc_sh.at[pl.ds(row, ZR)], msg_v.at[pl.ds(0, ZR)])
            pltpu.sync_copy(msg_v.at[pl.ds(0, ZR)], out_hbm.at[c, pl.ds(row, ZR)])

    return pl.kernel(
        body,
        out_type=jax.ShapeDtypeStruct((NC, N_PAD, D), jnp.float32),
        mesh=plsc.VectorSubcoreMesh(core_axis_name="c", subcore_axis_name="s",
                                    num_cores=NC, num_subcores=NS),
        scratch_types=[
            pltpu.VMEM((epw,), jnp.int32),
            pltpu.VMEM((c_sc, D), jnp.float32),
            pltpu.SemaphoreType.DMA,
            pltpu.VMEM_SHARED((N_PAD, D), jnp.float32),
        ],
    )


def _msg_body(et_ref, xsrc_ref, relw_ref, wflat_ref, msg_ref):
    et = et_ref[0, 0, :]
    onehot = (et[:, None] == lax.broadcasted_iota(jnp.int32, (E_BLK, NUM_RELS), 1)).astype(jnp.bfloat16)
    edw = jnp.dot(onehot, relw_ref[...], preferred_element_type=jnp.float32)
    x4 = jnp.concatenate([xsrc_ref[...]] * NUM_BASES, axis=1).astype(jnp.float32)
    ed4 = (x4 * edw).astype(jnp.bfloat16)
    msg_ref[...] = jnp.dot(ed4, wflat_ref[...], preferred_element_type=jnp.float32)


def _edge_messages(et3, xsrc, relw, wflat):
    n_e = xsrc.shape[0]
    nblk = n_e // E_BLK
    return pl.pallas_call(
        _msg_body,
        grid=(nblk,),
        in_specs=[
            pl.BlockSpec((1, 1, E_BLK), lambda i: (i, 0, 0)),
            pl.BlockSpec((E_BLK, D), lambda i: (i, 0)),
            pl.BlockSpec((NUM_RELS, NUM_BASES * D), lambda i: (0, 0)),
            pl.BlockSpec((NUM_BASES * D, D), lambda i: (0, 0)),
        ],
        out_specs=pl.BlockSpec((E_BLK, D), lambda i: (i, 0)),
        out_shape=jax.ShapeDtypeStruct((n_e, D), jnp.float32),
    )(et3, xsrc, relw, wflat)


def _node_body(part_ref, x_ref, wself_ref, rel_ref, wrel_ref, out_ref, relout_ref):
    i = pl.program_id(0)
    agg = jnp.sum(part_ref[...], axis=0)
    out_ref[...] = jnp.maximum(
        agg + jnp.dot(x_ref[...], wself_ref[...], preferred_element_type=jnp.float32), 0.0)

    @pl.when(i == 0)
    def _():
        ro = jnp.dot(rel_ref[...], wrel_ref[...], preferred_element_type=jnp.float32)
        row = lax.broadcasted_iota(jnp.int32, ro.shape, 0)
        relout_ref[...] = jnp.where(row == ro.shape[0] - 1, 0.0, ro)


def _node_update(partials, x, self_loop_weight, rel_emb, w_rel):
    nblk = (N_NODES + N_BLK - 1) // N_BLK
    p = partials.shape[0]
    return pl.pallas_call(
        _node_body,
        grid=(nblk,),
        in_specs=[
            pl.BlockSpec((p, N_BLK, D), lambda i: (0, i, 0)),
            pl.BlockSpec((N_BLK, D), lambda i: (i, 0)),
            pl.BlockSpec((D, D), lambda i: (0, 0)),
            pl.BlockSpec((NUM_RELS + 1, D), lambda i: (0, 0)),
            pl.BlockSpec((D, D), lambda i: (0, 0)),
        ],
        out_specs=[
            pl.BlockSpec((N_BLK, D), lambda i: (i, 0)),
            pl.BlockSpec((NUM_RELS + 1, D), lambda i: (0, 0)),
        ],
        out_shape=[
            jax.ShapeDtypeStruct((N_NODES, D), jnp.float32),
            jax.ShapeDtypeStruct((NUM_RELS + 1, D), jnp.float32),
        ],
    )(partials, x, self_loop_weight, rel_emb, w_rel)


def kernel(x, edge_index, edge_type, rel_emb, weight_basis, w_comp, self_loop_weight, w_rel):
    src = edge_index[0].astype(jnp.int32)
    dst = edge_index[1].astype(jnp.int32)
    et = edge_type.astype(jnp.int32)
    relw = (w_comp[:, :, None] * rel_emb[None, :NUM_RELS, :].transpose(1, 0, 2)).reshape(
        NUM_RELS, NUM_BASES * D).astype(jnp.bfloat16)
    wflat = weight_basis.reshape(NUM_BASES * D, D).astype(jnp.bfloat16)

    gather = _gather_kernel(E_HALF, C_GA)
    scatter = _scatter_kernel(E_HALF, C_SC)

    halves = []
    for h in range(2):
        sl = slice(h * E_HALF, (h + 1) * E_HALF)
        et3 = et[sl].reshape(E_HALF // E_BLK, 1, E_BLK)
        xsrc = gather(x, src[sl])
        msg = _edge_messages(et3, xsrc, relw, wflat)
        halves.append(scatter(msg, dst[sl]))

    partials = jnp.concatenate(halves, axis=0)
    node_repr, rel_out = _node_update(partials, x, self_loop_weight, rel_emb, w_rel)
    return node_repr, rel_out


# speedup vs baseline: 1.2550x; 1.0682x over previous
import functools

import jax
import jax.numpy as jnp
from jax import lax
from jax.experimental import pallas as pl
from jax.experimental.pallas import tpu as pltpu
from jax.experimental.pallas import tpu_sc as plsc

N_NODES = 10000
N_EDGES = 320000
D = 128
NUM_RELS = 64
NUM_BASES = 4

NC = 2
NS = 16
NW = NC * NS

E_HALF = N_EDGES // 2

E_BLK = 6400
N_BLK = 2048

C_GA = 400
C_SC = 160
N_PAD = 10240
NPT = N_PAD // NS
ZR = 128


@functools.cache
def _gather_kernel(n_e, c_ga):
    epw = n_e // NW
    nchunks = epw // c_ga

    def body(x_hbm, src_hbm, out_hbm,
             idx_all, rows0, rows1, gsem0, gsem1, osem0, osem1):
        wid = lax.axis_index("s") * NC + lax.axis_index("c")
        base = wid * epw
        rows = (rows0, rows1)
        gsem = (gsem0, gsem1)
        osem = (osem0, osem1)

        pltpu.sync_copy(src_hbm.at[pl.ds(base, epw)], idx_all)

        chunks = [(k * c_ga, c_ga) for k in range(epw // c_ga)]
        if epw % c_ga:
            chunks.append(((epw // c_ga) * c_ga, epw % c_ga))
        nchk = len(chunks)

        def gath(k, sl):
            off, sz = chunks[k]
            return pltpu.async_copy(x_hbm.at[idx_all.at[pl.ds(off, sz)]],
                                    rows[sl].at[pl.ds(0, sz)], gsem[sl])

        gathers = [gath(0, 0)]
        outs = [None] * nchk
        for k in range(nchk):
            s = k % 2
            if k + 1 < nchk:
                if k >= 1:
                    outs[k - 1].wait()
                gathers.append(gath(k + 1, 1 - s))
            gathers[k].wait()
            off, sz = chunks[k]
            outs[k] = pltpu.async_copy(
                rows[s].at[pl.ds(0, sz)], out_hbm.at[pl.ds(base + off, sz)],
                osem[s])
        if nchk >= 2:
            outs[nchk - 2].wait()
        outs[nchk - 1].wait()

    return pl.kernel(
        body,
        out_type=jax.ShapeDtypeStruct((n_e, D), jnp.float32),
        mesh=plsc.VectorSubcoreMesh(core_axis_name="c", subcore_axis_name="s",
                                    num_cores=NC, num_subcores=NS),
        scratch_types=[
            pltpu.VMEM((epw,), jnp.int32),
            pltpu.VMEM((c_ga, D), jnp.float32),
            pltpu.VMEM((c_ga, D), jnp.float32),
            pltpu.SemaphoreType.DMA,
            pltpu.SemaphoreType.DMA,
            pltpu.SemaphoreType.DMA,
            pltpu.SemaphoreType.DMA,
        ],
    )


@functools.cache
def _scatter_kernel(n_e, c_sc):
    epw = n_e // NW
    nchunks = epw // c_sc

    def body(msg_hbm, dst_hbm, out_hbm, idx_all, msg_v, msg_w,
             isem, msem0, msem1, acc_sh):
        c = lax.axis_index("c")
        s = lax.axis_index("s")
        wid = s * NC + c
        base = wid * epw

        idx_load = pltpu.async_copy(dst_hbm.at[pl.ds(base, epw)], idx_all, isem)

        def zrow(r, carry):
            for k in range(D // 16):
                msg_v[r, pl.ds(k * 16, 16)] = jnp.zeros((16,), jnp.float32)
            return carry

        lax.fori_loop(0, ZR, zrow, 0)
        for t in range(NPT // ZR):
            pltpu.sync_copy(msg_v.at[pl.ds(0, ZR)],
                            acc_sh.at[pl.ds(s * NPT + t * ZR, ZR)])
        idx_load.wait()
        plsc.subcore_barrier()

        chunks = [(k * c_sc, c_sc) for k in range(epw // c_sc)]
        if epw % c_sc:
            chunks.append(((epw // c_sc) * c_sc, epw % c_sc))
        msg = (msg_v, msg_w)
        msem = (msem0, msem1)

        def load(k, sl):
            off, sz = chunks[k]
            return pltpu.async_copy(msg_hbm.at[pl.ds(base + off, sz)],
                                    msg[sl].at[pl.ds(0, sz)], msem[sl])

        loads = [load(0, 0)]
        for k in range(len(chunks)):
            sl = k % 2
            if k + 1 < len(chunks):
                loads.append(load(k + 1, 1 - sl))
            off, sz = chunks[k]
            loads[k].wait()
            pltpu.sync_copy(msg[sl].at[pl.ds(0, sz)],
                            acc_sh.at[idx_all.at[pl.ds(off, sz)]], add=True)
        plsc.subcore_barrier()

        for t in range(NPT // ZR):
            row = s * NPT + t * ZR
            pltpu.sync_copy(acc_sh.at[pl.ds(row, ZR)], msg_v.at[pl.ds(0, ZR)])
            pltpu.sync_copy(msg_v.at[pl.ds(0, ZR)], out_hbm.at[c, pl.ds(row, ZR)])

    return pl.kernel(
        body,
        out_type=jax.ShapeDtypeStruct((NC, N_PAD, D), jnp.float32),
        mesh=plsc.VectorSubcoreMesh(core_axis_name="c", subcore_axis_name="s",
                                    num_cores=NC, num_subcores=NS),
        scratch_types=[
            pltpu.VMEM((epw,), jnp.int32),
            pltpu.VMEM((c_sc, D), jnp.float32),
            pltpu.VMEM((c_sc, D), jnp.float32),
            pltpu.SemaphoreType.DMA,
            pltpu.SemaphoreType.DMA,
            pltpu.SemaphoreType.DMA,
            pltpu.VMEM_SHARED((N_PAD, D), jnp.float32),
        ],
    )


def _msg_body(et_ref, xsrc_ref, relw_ref, wflat_ref, msg_ref):
    et = et_ref[0, 0, :]
    onehot = (et[:, None] == lax.broadcasted_iota(jnp.int32, (E_BLK, NUM_RELS), 1)).astype(jnp.bfloat16)
    edw = jnp.dot(onehot, relw_ref[...], preferred_element_type=jnp.float32)
    x4 = jnp.concatenate([xsrc_ref[...]] * NUM_BASES, axis=1).astype(jnp.float32)
    ed4 = (x4 * edw).astype(jnp.bfloat16)
    msg_ref[...] = jnp.dot(ed4, wflat_ref[...], preferred_element_type=jnp.float32)


def _edge_messages(et3, xsrc, relw, wflat):
    n_e = xsrc.shape[0]
    nblk = n_e // E_BLK
    return pl.pallas_call(
        _msg_body,
        grid=(nblk,),
        in_specs=[
            pl.BlockSpec((1, 1, E_BLK), lambda i: (i, 0, 0)),
            pl.BlockSpec((E_BLK, D), lambda i: (i, 0)),
            pl.BlockSpec((NUM_RELS, NUM_BASES * D), lambda i: (0, 0)),
            pl.BlockSpec((NUM_BASES * D, D), lambda i: (0, 0)),
        ],
        out_specs=pl.BlockSpec((E_BLK, D), lambda i: (i, 0)),
        out_shape=jax.ShapeDtypeStruct((n_e, D), jnp.float32),
    )(et3, xsrc, relw, wflat)


def _node_body(part_ref, x_ref, wself_ref, rel_ref, wrel_ref, out_ref, relout_ref):
    i = pl.program_id(0)
    agg = jnp.sum(part_ref[...], axis=0)
    out_ref[...] = jnp.maximum(
        agg + jnp.dot(x_ref[...], wself_ref[...], preferred_element_type=jnp.float32), 0.0)

    @pl.when(i == 0)
    def _():
        ro = jnp.dot(rel_ref[...], wrel_ref[...], preferred_element_type=jnp.float32)
        row = lax.broadcasted_iota(jnp.int32, ro.shape, 0)
        relout_ref[...] = jnp.where(row == ro.shape[0] - 1, 0.0, ro)


def _node_update(partials, x, self_loop_weight, rel_emb, w_rel):
    nblk = (N_NODES + N_BLK - 1) // N_BLK
    p = partials.shape[0]
    return pl.pallas_call(
        _node_body,
        grid=(nblk,),
        in_specs=[
            pl.BlockSpec((p, N_BLK, D), lambda i: (0, i, 0)),
            pl.BlockSpec((N_BLK, D), lambda i: (i, 0)),
            pl.BlockSpec((D, D), lambda i: (0, 0)),
            pl.BlockSpec((NUM_RELS + 1, D), lambda i: (0, 0)),
            pl.BlockSpec((D, D), lambda i: (0, 0)),
        ],
        out_specs=[
            pl.BlockSpec((N_BLK, D), lambda i: (i, 0)),
            pl.BlockSpec((NUM_RELS + 1, D), lambda i: (0, 0)),
        ],
        out_shape=[
            jax.ShapeDtypeStruct((N_NODES, D), jnp.float32),
            jax.ShapeDtypeStruct((NUM_RELS + 1, D), jnp.float32),
        ],
    )(partials, x, self_loop_weight, rel_emb, w_rel)


def kernel(x, edge_index, edge_type, rel_emb, weight_basis, w_comp, self_loop_weight, w_rel):
    src = edge_index[0].astype(jnp.int32)
    dst = edge_index[1].astype(jnp.int32)
    et = edge_type.astype(jnp.int32)
    relw = (w_comp[:, :, None] * rel_emb[None, :NUM_RELS, :].transpose(1, 0, 2)).reshape(
        NUM_RELS, NUM_BASES * D).astype(jnp.bfloat16)
    wflat = weight_basis.reshape(NUM_BASES * D, D).astype(jnp.bfloat16)

    gather = _gather_kernel(E_HALF, C_GA)
    scatter = _scatter_kernel(E_HALF, C_SC)

    halves = []
    for h in range(2):
        sl = slice(h * E_HALF, (h + 1) * E_HALF)
        et3 = et[sl].reshape(E_HALF // E_BLK, 1, E_BLK)
        xsrc = gather(x, src[sl])
        msg = _edge_messages(et3, xsrc, relw, wflat)
        halves.append(scatter(msg, dst[sl]))

    partials = jnp.concatenate(halves, axis=0)
    node_repr, rel_out = _node_update(partials, x, self_loop_weight, rel_emb, w_rel)
    return node_repr, rel_out


# speedup vs baseline: 1.2582x; 1.0025x over previous
import functools

import jax
import jax.numpy as jnp
from jax import lax
from jax.experimental import pallas as pl
from jax.experimental.pallas import tpu as pltpu
from jax.experimental.pallas import tpu_sc as plsc

N_NODES = 10000
N_EDGES = 320000
D = 128
NUM_RELS = 64
NUM_BASES = 4

NC = 2
NS = 16
NW = NC * NS

E_HALF = N_EDGES // 2

E_BLK = 6400
N_BLK = 2048

C_GA = 400
C_SC = 160
N_PAD = 10240
NPT = N_PAD // NS
ZR = 128


@functools.cache
def _gather_kernel(n_e, c_ga):
    epw = n_e // NW
    nchunks = epw // c_ga

    def body(x_hbm, src_hbm, out_hbm,
             idx_all, rows0, rows1, gsem0, gsem1, osem0, osem1):
        wid = lax.axis_index("s") * NC + lax.axis_index("c")
        base = wid * epw
        rows = (rows0, rows1)
        gsem = (gsem0, gsem1)
        osem = (osem0, osem1)

        pltpu.sync_copy(src_hbm.at[pl.ds(base, epw)], idx_all)

        chunks = [(k * c_ga, c_ga) for k in range(epw // c_ga)]
        if epw % c_ga:
            chunks.append(((epw // c_ga) * c_ga, epw % c_ga))
        nchk = len(chunks)

        def gath(k, sl):
            off, sz = chunks[k]
            return pltpu.async_copy(x_hbm.at[idx_all.at[pl.ds(off, sz)]],
                                    rows[sl].at[pl.ds(0, sz)], gsem[sl])

        gathers = [gath(0, 0)]
        outs = [None] * nchk
        for k in range(nchk):
            s = k % 2
            if k + 1 < nchk:
                if k >= 1:
                    outs[k - 1].wait()
                gathers.append(gath(k + 1, 1 - s))
            gathers[k].wait()
            off, sz = chunks[k]
            outs[k] = pltpu.async_copy(
                rows[s].at[pl.ds(0, sz)], out_hbm.at[pl.ds(base + off, sz)],
                osem[s])
        if nchk >= 2:
            outs[nchk - 2].wait()
        outs[nchk - 1].wait()

    return pl.kernel(
        body,
        out_type=jax.ShapeDtypeStruct((n_e, D), jnp.float32),
        mesh=plsc.VectorSubcoreMesh(core_axis_name="c", subcore_axis_name="s",
                                    num_cores=NC, num_subcores=NS),
        scratch_types=[
            pltpu.VMEM((epw,), jnp.int32),
            pltpu.VMEM((c_ga, D), jnp.float32),
            pltpu.VMEM((c_ga, D), jnp.float32),
            pltpu.SemaphoreType.DMA,
            pltpu.SemaphoreType.DMA,
            pltpu.SemaphoreType.DMA,
            pltpu.SemaphoreType.DMA,
        ],
    )


@functools.cache
def _scatter_kernel(n_e, c_sc):
    epw = n_e // NW
    nchunks = epw // c_sc

    def body(msg_hbm, dst_hbm, out_hbm, idx_all, msg_v, msg_w,
             isem, msem0, msem1, acc_sh):
        c = lax.axis_index("c")
        s = lax.axis_index("s")
        wid = s * NC + c
        base = wid * epw

        idx_load = pltpu.async_copy(dst_hbm.at[pl.ds(base, epw)], idx_all, isem)

        def zrow(r, carry):
            for k in range(D // 16):
                msg_v[r, pl.ds(k * 16, 16)] = jnp.zeros((16,), jnp.float32)
            return carry

        lax.fori_loop(0, ZR, zrow, 0)
        for t in range(NPT // ZR):
            pltpu.sync_copy(msg_v.at[pl.ds(0, ZR)],
                            acc_sh.at[pl.ds(s * NPT + t * ZR, ZR)])
        idx_load.wait()
        plsc.subcore_barrier()

        chunks = [(k * c_sc, c_sc) for k in range(epw // c_sc)]
        if epw % c_sc:
            chunks.append(((epw // c_sc) * c_sc, epw % c_sc))
        msg = (msg_v, msg_w)
        msem = (msem0, msem1)

        def load(k, sl):
            off, sz = chunks[k]
            return pltpu.async_copy(msg_hbm.at[pl.ds(base + off, sz)],
                                    msg[sl].at[pl.ds(0, sz)], msem[sl])

        loads = [load(0, 0)]
        for k in range(len(chunks)):
            sl = k % 2
            if k + 1 < len(chunks):
                loads.append(load(k + 1, 1 - sl))
            off, sz = chunks[k]
            loads[k].wait()
            pltpu.sync_copy(msg[sl].at[pl.ds(0, sz)],
                            acc_sh.at[idx_all.at[pl.ds(off, sz)]], add=True)
        plsc.subcore_barrier()

        pltpu.sync_copy(acc_sh.at[pl.ds(s * NPT, NPT)],
                        out_hbm.at[c, pl.ds(s * NPT, NPT)])

    return pl.kernel(
        body,
        out_type=jax.ShapeDtypeStruct((NC, N_PAD, D), jnp.float32),
        mesh=plsc.VectorSubcoreMesh(core_axis_name="c", subcore_axis_name="s",
                                    num_cores=NC, num_subcores=NS),
        scratch_types=[
            pltpu.VMEM((epw,), jnp.int32),
            pltpu.VMEM((c_sc, D), jnp.float32),
            pltpu.VMEM((c_sc, D), jnp.float32),
            pltpu.SemaphoreType.DMA,
            pltpu.SemaphoreType.DMA,
            pltpu.SemaphoreType.DMA,
            pltpu.VMEM_SHARED((N_PAD, D), jnp.float32),
        ],
    )


def _msg_body(et_ref, xsrc_ref, relw_ref, wflat_ref, msg_ref):
    et = et_ref[0, 0, :]
    onehot = (et[:, None] == lax.broadcasted_iota(jnp.int32, (E_BLK, NUM_RELS), 1)).astype(jnp.bfloat16)
    edw = jnp.dot(onehot, relw_ref[...], preferred_element_type=jnp.float32)
    x4 = jnp.concatenate([xsrc_ref[...]] * NUM_BASES, axis=1).astype(jnp.float32)
    ed4 = (x4 * edw).astype(jnp.bfloat16)
    msg_ref[...] = jnp.dot(ed4, wflat_ref[...], preferred_element_type=jnp.float32)


def _edge_messages(et3, xsrc, relw, wflat):
    n_e = xsrc.shape[0]
    nblk = n_e // E_BLK
    return pl.pallas_call(
        _msg_body,
        grid=(nblk,),
        in_specs=[
            pl.BlockSpec((1, 1, E_BLK), lambda i: (i, 0, 0)),
            pl.BlockSpec((E_BLK, D), lambda i: (i, 0)),
            pl.BlockSpec((NUM_RELS, NUM_BASES * D), lambda i: (0, 0)),
            pl.BlockSpec((NUM_BASES * D, D), lambda i: (0, 0)),
        ],
        out_specs=pl.BlockSpec((E_BLK, D), lambda i: (i, 0)),
        out_shape=jax.ShapeDtypeStruct((n_e, D), jnp.float32),
    )(et3, xsrc, relw, wflat)


def _node_body(part_ref, x_ref, wself_ref, rel_ref, wrel_ref, out_ref, relout_ref):
    i = pl.program_id(0)
    agg = jnp.sum(part_ref[...], axis=0)
    out_ref[...] = jnp.maximum(
        agg + jnp.dot(x_ref[...], wself_ref[...], preferred_element_type=jnp.float32), 0.0)

    @pl.when(i == 0)
    def _():
        ro = jnp.dot(rel_ref[...], wrel_ref[...], preferred_element_type=jnp.float32)
        row = lax.broadcasted_iota(jnp.int32, ro.shape, 0)
        relout_ref[...] = jnp.where(row == ro.shape[0] - 1, 0.0, ro)


def _node_update(partials, x, self_loop_weight, rel_emb, w_rel):
    nblk = (N_NODES + N_BLK - 1) // N_BLK
    p = partials.shape[0]
    return pl.pallas_call(
        _node_body,
        grid=(nblk,),
        in_specs=[
            pl.BlockSpec((p, N_BLK, D), lambda i: (0, i, 0)),
            pl.BlockSpec((N_BLK, D), lambda i: (i, 0)),
            pl.BlockSpec((D, D), lambda i: (0, 0)),
            pl.BlockSpec((NUM_RELS + 1, D), lambda i: (0, 0)),
            pl.BlockSpec((D, D), lambda i: (0, 0)),
        ],
        out_specs=[
            pl.BlockSpec((N_BLK, D), lambda i: (i, 0)),
            pl.BlockSpec((NUM_RELS + 1, D), lambda i: (0, 0)),
        ],
        out_shape=[
            jax.ShapeDtypeStruct((N_NODES, D), jnp.float32),
            jax.ShapeDtypeStruct((NUM_RELS + 1, D), jnp.float32),
        ],
    )(partials, x, self_loop_weight, rel_emb, w_rel)


def kernel(x, edge_index, edge_type, rel_emb, weight_basis, w_comp, self_loop_weight, w_rel):
    src = edge_index[0].astype(jnp.int32)
    dst = edge_index[1].astype(jnp.int32)
    et = edge_type.astype(jnp.int32)
    relw = (w_comp[:, :, None] * rel_emb[None, :NUM_RELS, :].transpose(1, 0, 2)).reshape(
        NUM_RELS, NUM_BASES * D).astype(jnp.bfloat16)
    wflat = weight_basis.reshape(NUM_BASES * D, D).astype(jnp.bfloat16)

    gather = _gather_kernel(E_HALF, C_GA)
    scatter = _scatter_kernel(E_HALF, C_SC)

    halves = []
    for h in range(2):
        sl = slice(h * E_HALF, (h + 1) * E_HALF)
        et3 = et[sl].reshape(E_HALF // E_BLK, 1, E_BLK)
        xsrc = gather(x, src[sl])
        msg = _edge_messages(et3, xsrc, relw, wflat)
        halves.append(scatter(msg, dst[sl]))

    partials = jnp.concatenate(halves, axis=0)
    node_repr, rel_out = _node_update(partials, x, self_loop_weight, rel_emb, w_rel)
    return node_repr, rel_out
